# baseline (speedup 1.0000x reference)
import jax
import jax.numpy as jnp
from jax.experimental import pallas as pl
from jax.experimental.pallas import tpu as pltpu

_BB = 512
_VMEM_LIMIT = 100 * 1024 * 1024


def _fused_kernel(x_ref, w1_ref, b1_ref, w2_ref, b2_ref,
                  fc1_ref, fb1_ref, fc2_ref, fb2_ref, o_ref):
    bb = x_ref.shape[0]

    x = x_ref[...]
    xa = jnp.concatenate([x[:, j:j + 6, :] for j in range(2)], axis=-1)
    xa = xa.reshape(bb * 6, 224)
    y1 = jnp.dot(xa, w1_ref[...], preferred_element_type=jnp.float32)
    y1 = y1.reshape(bb, 6, 3072)

    m = jnp.maximum(y1[:, :, :1536], y1[:, :, 1536:])
    m = jnp.maximum(m[:, :, :768], m[:, :, 768:])
    p1 = jnp.maximum(m + b1_ref[...], 0.0)

    xb = jnp.concatenate([p1[:, j:j + 4, :] for j in range(3)], axis=-1)
    xb = xb.reshape(bb * 4, 2304)
    y2 = jnp.dot(xb, w2_ref[...], preferred_element_type=jnp.float32)
    y2 = y2.reshape(bb, 4, 1024)

    m2 = jnp.maximum(y2[:, :, :512], y2[:, :, 512:])
    m2 = jnp.maximum(m2[:, :, :256], m2[:, :, 256:])
    p2 = jnp.maximum(m2 + b2_ref[...], 0.0)

    hh = jnp.dot(p2[:, 0, :], fc1_ref[0], preferred_element_type=jnp.float32)
    for ph in range(1, 4):
        hh = hh + jnp.dot(p2[:, ph, :], fc1_ref[ph],
                          preferred_element_type=jnp.float32)
    hh = jnp.maximum(hh + fb1_ref[...], 0.0)

    logits = jnp.dot(hh, fc2_ref[...], preferred_element_type=jnp.float32)
    logits = logits + fb2_ref[...]
    mx = jnp.max(logits, axis=-1, keepdims=True)
    e = jnp.exp(logits - mx)
    o_ref[...] = (e / jnp.sum(e, axis=-1, keepdims=True)).astype(o_ref.dtype)


def _band_weights(conv1_w, conv2_w):
    w1r = conv1_w.reshape(5, 5, 32)
    kh = jnp.arange(5)
    rh1 = (jnp.arange(8)[None, :, None, None]
           - 2 * jnp.arange(2)[None, None, :, None]
           - jnp.arange(2)[None, None, None, :]) == kh[:, None, None, None]
    rh1 = rh1.astype(jnp.float32)
    rw1 = (jnp.arange(28)[None, :, None, None]
           - 2 * jnp.arange(12)[None, None, None, :]
           - jnp.arange(2)[None, None, :, None]) == kh[:, None, None, None]
    rw1 = rw1.astype(jnp.float32)
    W1 = jnp.einsum('hwc,hdpq,wiur->diqupcr', w1r, rh1, rw1)
    W1 = W1.reshape(224, 3072)

    w2v = conv2_w.reshape(32, 5, 5, 64)
    rh2 = (jnp.arange(6)[None, :, None]
           - jnp.arange(2)[None, None, :]) == kh[:, None, None]
    rh2 = rh2.astype(jnp.float32)
    rw2 = (jnp.arange(12)[None, :, None, None]
           - 2 * jnp.arange(4)[None, None, None, :]
           - jnp.arange(2)[None, None, :, None]) == kh[:, None, None, None]
    rw2 = rw2.astype(jnp.float32)
    W2 = jnp.einsum('ghwc,hsq,wiur->sgiqucr', w2v, rh2, rw2)
    W2 = W2.reshape(2304, 1024)
    return W1, W2


def kernel(x, conv1_w, conv1_b, conv2_w, conv2_b, fc1_w, fc1_b, fc2_w, fc2_b):
    n = x.shape[0]
    xr = x.reshape(n, 7, 112)
    W1, W2 = _band_weights(conv1_w, conv2_w)
    b1 = jnp.tile(jnp.repeat(conv1_b[0], 12), 2).reshape(1, 768)
    b2 = jnp.repeat(conv2_b[0], 4).reshape(1, 256)
    fc1p = fc1_w.reshape(4, 4, 64, 128).transpose(0, 2, 1, 3).reshape(4, 256, 128)

    bb = _BB if n % _BB == 0 else n
    grid = (n // bb,)
    return pl.pallas_call(
        _fused_kernel,
        out_shape=jax.ShapeDtypeStruct((n, 10), x.dtype),
        grid=grid,
        in_specs=[
            pl.BlockSpec((bb, 7, 112), lambda i: (i, 0, 0)),
            pl.BlockSpec((224, 3072), lambda i: (0, 0)),
            pl.BlockSpec((1, 768), lambda i: (0, 0)),
            pl.BlockSpec((2304, 1024), lambda i: (0, 0)),
            pl.BlockSpec((1, 256), lambda i: (0, 0)),
            pl.BlockSpec((4, 256, 128), lambda i: (0, 0, 0)),
            pl.BlockSpec((1, 128), lambda i: (0, 0)),
            pl.BlockSpec((128, 10), lambda i: (0, 0)),
            pl.BlockSpec((1, 10), lambda i: (0, 0)),
        ],
        out_specs=pl.BlockSpec((bb, 10), lambda i: (i, 0)),
        compiler_params=pltpu.CompilerParams(
            dimension_semantics=("parallel",),
            vmem_limit_bytes=_VMEM_LIMIT,
        ),
        cost_estimate=pl.CostEstimate(
            flops=2 * n * (6 * 224 * 3072 + 4 * 2304 * 1024 + 1024 * 128 + 128 * 10),
            transcendentals=n * 10,
            bytes_accessed=4 * (n * 28 * 28 + n * 10),
        ),
    )(xr, W1, b1, W2, b2, fc1p, fc1_b, fc2_w, fc2_b)


# speedup vs baseline: 1.1744x; 1.1744x over previous
import jax
import jax.numpy as jnp
from jax.experimental import pallas as pl
from jax.experimental.pallas import tpu as pltpu

_BB = 256
_VMEM_LIMIT = 100 * 1024 * 1024


def _fused_kernel(x_ref, w1_ref, b1_ref, w2_ref, b2_ref,
                  fc1_ref, fb1_ref, fc2_ref, fb2_ref, o_ref):
    bb = x_ref.shape[0]

    x = x_ref[...]
    xa = jnp.concatenate([x[:, j:j + 6, :] for j in range(2)], axis=-1)
    xa = xa.reshape(bb * 6, 224)
    y1 = jnp.dot(xa, w1_ref[...], preferred_element_type=jnp.float32)
    y1 = y1.reshape(bb, 6, 3072)

    m = jnp.maximum(y1[:, :, :1536], y1[:, :, 1536:])
    m = jnp.maximum(m[:, :, :768], m[:, :, 768:])
    p1 = jnp.maximum(m + b1_ref[...], 0.0)

    xb = jnp.concatenate([p1[:, j:j + 4, :] for j in range(3)], axis=-1)
    xb = xb.reshape(bb * 4, 2304)
    y2 = jnp.dot(xb, w2_ref[...], preferred_element_type=jnp.float32)
    y2 = y2.reshape(bb, 4, 1024)

    m2 = jnp.maximum(y2[:, :, :512], y2[:, :, 512:])
    m2 = jnp.maximum(m2[:, :, :256], m2[:, :, 256:])
    p2 = jnp.maximum(m2 + b2_ref[...], 0.0)

    hh = jnp.dot(p2[:, 0, :], fc1_ref[0], preferred_element_type=jnp.float32)
    for ph in range(1, 4):
        hh = hh + jnp.dot(p2[:, ph, :], fc1_ref[ph],
                          preferred_element_type=jnp.float32)
    hh = jnp.maximum(hh + fb1_ref[...], 0.0)

    logits = jnp.dot(hh, fc2_ref[...], preferred_element_type=jnp.float32)
    logits = logits + fb2_ref[...]
    mx = jnp.max(logits, axis=-1, keepdims=True)
    e = jnp.exp(logits - mx)
    o_ref[...] = (e / jnp.sum(e, axis=-1, keepdims=True)).astype(o_ref.dtype)


def _band_weights(conv1_w, conv2_w):
    w1r = conv1_w.reshape(5, 5, 32)
    kh = jnp.arange(5)
    rh1 = (jnp.arange(8)[None, :, None, None]
           - 2 * jnp.arange(2)[None, None, :, None]
           - jnp.arange(2)[None, None, None, :]) == kh[:, None, None, None]
    rh1 = rh1.astype(jnp.float32)
    rw1 = (jnp.arange(28)[None, :, None, None]
           - 2 * jnp.arange(12)[None, None, None, :]
           - jnp.arange(2)[None, None, :, None]) == kh[:, None, None, None]
    rw1 = rw1.astype(jnp.float32)
    W1 = jnp.einsum('hwc,hdpq,wiur->diquprc', w1r, rh1, rw1)
    W1 = W1.reshape(224, 3072)

    w2v = conv2_w.reshape(32, 5, 5, 64)
    rh2 = (jnp.arange(6)[None, :, None]
           - jnp.arange(2)[None, None, :]) == kh[:, None, None]
    rh2 = rh2.astype(jnp.float32)
    rw2 = (jnp.arange(12)[None, :, None, None]
           - 2 * jnp.arange(4)[None, None, None, :]
           - jnp.arange(2)[None, None, :, None]) == kh[:, None, None, None]
    rw2 = rw2.astype(jnp.float32)
    W2 = jnp.einsum('ghwc,hsq,wiur->sigqurc', w2v, rh2, rw2)
    W2 = W2.reshape(2304, 1024)
    return W1, W2


def kernel(x, conv1_w, conv1_b, conv2_w, conv2_b, fc1_w, fc1_b, fc2_w, fc2_b):
    n = x.shape[0]
    xr = x.reshape(n, 7, 112)
    W1, W2 = _band_weights(conv1_w, conv2_w)
    b1 = jnp.tile(conv1_b[0], 24).reshape(1, 768)
    b2 = jnp.tile(conv2_b[0], 4).reshape(1, 256)
    fc1p = fc1_w.reshape(4, 256, 128)

    bb = _BB if n % _BB == 0 else n
    grid = (n // bb,)
    return pl.pallas_call(
        _fused_kernel,
        out_shape=jax.ShapeDtypeStruct((n, 10), x.dtype),
        grid=grid,
        in_specs=[
            pl.BlockSpec((bb, 7, 112), lambda i: (i, 0, 0)),
            pl.BlockSpec((224, 3072), lambda i: (0, 0)),
            pl.BlockSpec((1, 768), lambda i: (0, 0)),
            pl.BlockSpec((2304, 1024), lambda i: (0, 0)),
            pl.BlockSpec((1, 256), lambda i: (0, 0)),
            pl.BlockSpec((4, 256, 128), lambda i: (0, 0, 0)),
            pl.BlockSpec((1, 128), lambda i: (0, 0)),
            pl.BlockSpec((128, 10), lambda i: (0, 0)),
            pl.BlockSpec((1, 10), lambda i: (0, 0)),
        ],
        out_specs=pl.BlockSpec((bb, 10), lambda i: (i, 0)),
        compiler_params=pltpu.CompilerParams(
            dimension_semantics=("parallel",),
            vmem_limit_bytes=_VMEM_LIMIT,
        ),
        cost_estimate=pl.CostEstimate(
            flops=2 * n * (6 * 224 * 3072 + 4 * 2304 * 1024 + 1024 * 128 + 128 * 10),
            transcendentals=n * 10,
            bytes_accessed=4 * (n * 28 * 28 + n * 10),
        ),
    )(xr, W1, b1, W2, b2, fc1p, fc1_b, fc2_w, fc2_b)


# speedup vs baseline: 1.3979x; 1.1904x over previous
import jax
import jax.numpy as jnp
from jax.experimental import pallas as pl
from jax.experimental.pallas import tpu as pltpu

_BB = 256
_VMEM_LIMIT = 100 * 1024 * 1024


def _fused_kernel(x_ref, w1_ref, b1_ref, w2_ref, b2_ref,
                  fc1_ref, fb1_ref, fc2_ref, fb2_ref, o_ref):
    bb = x_ref.shape[1]

    x = x_ref[...]
    xa = jnp.concatenate([x[0:6], x[1:7]], axis=-1)
    xa = xa.reshape(6 * bb, 224)
    y1 = jnp.dot(xa, w1_ref[...], preferred_element_type=jnp.float32)
    y1 = y1.reshape(6, bb, 3072)

    m = jnp.maximum(y1[:, :, :1536], y1[:, :, 1536:])
    m = jnp.maximum(m[:, :, :768], m[:, :, 768:])
    p1 = jnp.maximum(m + b1_ref[...], 0.0)

    xb = jnp.concatenate([p1[0:4], p1[1:5], p1[2:6]], axis=-1)
    xb = xb.reshape(4 * bb, 2304)
    y2 = jnp.dot(xb, w2_ref[...], preferred_element_type=jnp.float32)
    y2 = y2.reshape(4, bb, 1024)

    m2 = jnp.maximum(y2[:, :, :512], y2[:, :, 512:])
    m2 = jnp.maximum(m2[:, :, :256], m2[:, :, 256:])
    p2 = jnp.maximum(m2 + b2_ref[...], 0.0)

    hh = jnp.dot(p2[0], fc1_ref[0], preferred_element_type=jnp.float32)
    for ph in range(1, 4):
        hh = hh + jnp.dot(p2[ph], fc1_ref[ph],
                          preferred_element_type=jnp.float32)
    hh = jnp.maximum(hh + fb1_ref[...], 0.0)

    logits = jnp.dot(hh, fc2_ref[...], preferred_element_type=jnp.float32)
    logits = logits + fb2_ref[...]
    mx = jnp.max(logits, axis=-1, keepdims=True)
    e = jnp.exp(logits - mx)
    o_ref[...] = (e / jnp.sum(e, axis=-1, keepdims=True)).astype(o_ref.dtype)


def _band_weights(conv1_w, conv2_w):
    w1r = conv1_w.reshape(5, 5, 32)
    kh = jnp.arange(5)
    rh1 = (jnp.arange(8)[None, :, None, None]
           - 2 * jnp.arange(2)[None, None, :, None]
           - jnp.arange(2)[None, None, None, :]) == kh[:, None, None, None]
    rh1 = rh1.astype(jnp.float32)
    rw1 = (jnp.arange(28)[None, :, None, None]
           - 2 * jnp.arange(12)[None, None, None, :]
           - jnp.arange(2)[None, None, :, None]) == kh[:, None, None, None]
    rw1 = rw1.astype(jnp.float32)
    W1 = jnp.einsum('hwc,hdpq,wiur->diquprc', w1r, rh1, rw1)
    W1 = W1.reshape(224, 3072)

    w2v = conv2_w.reshape(32, 5, 5, 64)
    rh2 = (jnp.arange(6)[None, :, None]
           - jnp.arange(2)[None, None, :]) == kh[:, None, None]
    rh2 = rh2.astype(jnp.float32)
    rw2 = (jnp.arange(12)[None, :, None, None]
           - 2 * jnp.arange(4)[None, None, None, :]
           - jnp.arange(2)[None, None, :, None]) == kh[:, None, None, None]
    rw2 = rw2.astype(jnp.float32)
    W2 = jnp.einsum('ghwc,hsq,wiur->sigqurc', w2v, rh2, rw2)
    W2 = W2.reshape(2304, 1024)
    return W1, W2


def kernel(x, conv1_w, conv1_b, conv2_w, conv2_b, fc1_w, fc1_b, fc2_w, fc2_b):
    n = x.shape[0]
    xr = x.reshape(n, 7, 112).transpose(1, 0, 2)
    W1, W2 = _band_weights(conv1_w, conv2_w)
    b1 = jnp.tile(conv1_b[0], 24).reshape(1, 768)
    b2 = jnp.tile(conv2_b[0], 4).reshape(1, 256)
    fc1p = fc1_w.reshape(4, 256, 128)

    bb = _BB if n % _BB == 0 else n
    grid = (n // bb,)
    return pl.pallas_call(
        _fused_kernel,
        out_shape=jax.ShapeDtypeStruct((n, 10), x.dtype),
        grid=grid,
        in_specs=[
            pl.BlockSpec((7, bb, 112), lambda i: (0, i, 0)),
            pl.BlockSpec((224, 3072), lambda i: (0, 0)),
            pl.BlockSpec((1, 768), lambda i: (0, 0)),
            pl.BlockSpec((2304, 1024), lambda i: (0, 0)),
            pl.BlockSpec((1, 256), lambda i: (0, 0)),
            pl.BlockSpec((4, 256, 128), lambda i: (0, 0, 0)),
            pl.BlockSpec((1, 128), lambda i: (0, 0)),
            pl.BlockSpec((128, 10), lambda i: (0, 0)),
            pl.BlockSpec((1, 10), lambda i: (0, 0)),
        ],
        out_specs=pl.BlockSpec((bb, 10), lambda i: (i, 0)),
        compiler_params=pltpu.CompilerParams(
            dimension_semantics=("parallel",),
            vmem_limit_bytes=_VMEM_LIMIT,
        ),
        cost_estimate=pl.CostEstimate(
            flops=2 * n * (6 * 224 * 3072 + 4 * 2304 * 1024 + 1024 * 128 + 128 * 10),
            transcendentals=n * 10,
            bytes_accessed=4 * (n * 28 * 28 + n * 10),
        ),
    )(xr, W1, b1, W2, b2, fc1p, fc1_b, fc2_w, fc2_b)
